# Initial kernel scaffold; baseline (speedup 1.0000x reference)
import jax
import jax.numpy as jnp
from jax.experimental import pallas as pl


def kernel(x, edge_index, edge_attr, W1, b1, W2, b2, W3, b3):
    raise NotImplementedError("write your pallas kernel here")


# speedup vs baseline: 33.3584x; 33.3584x over previous
import functools

import jax
import jax.numpy as jnp
from jax import lax
from jax.experimental import pallas as pl
from jax.experimental.pallas import tpu as pltpu
from jax.experimental.pallas import tpu_sc as plsc

N = 10000
NP = 10240
FH = 16
NC = 2
NS = 16
NW = NC * NS
CH = 128
GPT = 80
EPT = CH * GPT
E_PAD = NW * EPT
NROWS = E_PAD // CH
N_PER_S = NP // NS

_mesh = plsc.VectorSubcoreMesh(
    core_axis_name="c", subcore_axis_name="s", num_cores=NC, num_subcores=NS)


def _worker_id():
    return lax.axis_index("s") * NC + lax.axis_index("c")


@functools.partial(
    pl.kernel,
    out_type=jax.ShapeDtypeStruct((NW * NP,), jnp.float32),
    mesh=_mesh,
    compiler_params=pltpu.CompilerParams(needs_layout_passes=False, use_tc_tiling_on_sc=False),
    scratch_types=[
        pltpu.VMEM((GPT, CH), jnp.int32),
        pltpu.VMEM((GPT, CH), jnp.float32),
        pltpu.VMEM((NP,), jnp.float32),
    ],
)
def _sc_deg(dst_hbm, ew_hbm, out_hbm, dstv, ewv, accv):
    wid = _worker_id()
    pltpu.sync_copy(dst_hbm.at[pl.ds(wid * GPT, GPT)], dstv)
    pltpu.sync_copy(ew_hbm.at[pl.ds(wid * GPT, GPT)], ewv)
    z = jnp.zeros((16,), jnp.float32)

    def zb(i, carry):
        accv[pl.ds(i * 16, 16)] = z
        return carry
    lax.fori_loop(0, NP // 16, zb, 0, unroll=8)

    def eb(g, carry):
        def ib(k, c2):
            idx = dstv[g, pl.ds(k * 16, 16)]
            w = ewv[g, pl.ds(k * 16, 16)]
            plsc.addupdate_scatter(accv, [idx], w)
            return c2
        return lax.fori_loop(0, CH // 16, ib, carry, unroll=8)
    lax.fori_loop(0, GPT, eb, 0)
    pltpu.sync_copy(accv, out_hbm.at[pl.ds(wid * NP, NP)])


@functools.partial(
    pl.kernel,
    out_type=jax.ShapeDtypeStruct((NC, NP, FH), jnp.float32),
    mesh=_mesh,
    compiler_params=pltpu.CompilerParams(needs_layout_passes=False, use_tc_tiling_on_sc=False),
    scratch_types=[
        pltpu.VMEM((GPT, CH), jnp.int32),
        pltpu.VMEM((GPT, CH), jnp.int32),
        pltpu.VMEM((GPT, CH), jnp.float32),
        pltpu.VMEM((CH, FH), jnp.float32),
        pltpu.VMEM_SHARED((NP, FH), jnp.float32),
        pltpu.SemaphoreType.DMA,
    ],
)
def _sc_edge16(src_hbm, dst_hbm, ew_hbm, hp_hbm, out_hbm,
               srcv, dstv, ewv, rows, accs, sem):
    s = lax.axis_index("s")
    c = lax.axis_index("c")
    wid = s * NC + c
    pltpu.sync_copy(src_hbm.at[pl.ds(wid * GPT, GPT)], srcv)
    pltpu.sync_copy(dst_hbm.at[pl.ds(wid * GPT, GPT)], dstv)
    pltpu.sync_copy(ew_hbm.at[pl.ds(wid * GPT, GPT)], ewv)

    z = jnp.zeros((16,), jnp.float32)

    def zb(i, carry):
        rows[i] = z
        return carry
    lax.fori_loop(0, CH, zb, 0, unroll=8)
    for q in range(N_PER_S // CH):
        pltpu.sync_copy(rows, accs.at[pl.ds(s * N_PER_S + q * CH, CH)])
    plsc.subcore_barrier()

    def gb(g, carry):
        pltpu.async_copy(hp_hbm.at[srcv.at[g]], rows, sem).wait()

        def mb(j, c2):
            wv = ewv[g, pl.ds(j * 16, 16)]
            for t in range(16):
                rows[j * 16 + t] = rows[j * 16 + t] * wv[t]
            return c2
        lax.fori_loop(0, CH // 16, mb, 0)
        pltpu.sync_copy(rows, accs.at[dstv.at[g]], add=True)
        return carry
    lax.fori_loop(0, GPT, gb, 0)

    plsc.subcore_barrier()
    pltpu.sync_copy(accs.at[pl.ds(s * N_PER_S, N_PER_S)],
                    out_hbm.at[c, pl.ds(s * N_PER_S, N_PER_S)])


@functools.partial(
    pl.kernel,
    out_type=jax.ShapeDtypeStruct((NW * NP,), jnp.float32),
    mesh=_mesh,
    compiler_params=pltpu.CompilerParams(needs_layout_passes=False, use_tc_tiling_on_sc=False),
    scratch_types=[
        pltpu.VMEM((GPT, CH), jnp.int32),
        pltpu.VMEM((GPT, CH), jnp.int32),
        pltpu.VMEM((GPT, CH), jnp.float32),
        pltpu.VMEM((NP,), jnp.float32),
        pltpu.VMEM((NP,), jnp.float32),
    ],
)
def _sc_edge1(src_hbm, dst_hbm, ew_hbm, h3_hbm, out_hbm,
              srcv, dstv, ewv, hv, accv):
    wid = _worker_id()
    pltpu.sync_copy(src_hbm.at[pl.ds(wid * GPT, GPT)], srcv)
    pltpu.sync_copy(dst_hbm.at[pl.ds(wid * GPT, GPT)], dstv)
    pltpu.sync_copy(ew_hbm.at[pl.ds(wid * GPT, GPT)], ewv)
    pltpu.sync_copy(h3_hbm, hv)
    z = jnp.zeros((16,), jnp.float32)

    def zb(i, carry):
        accv[pl.ds(i * 16, 16)] = z
        return carry
    lax.fori_loop(0, NP // 16, zb, 0, unroll=8)

    def eb(g, carry):
        def ib(k, c2):
            sl = pl.ds(k * 16, 16)
            vals = plsc.load_gather(hv, [srcv[g, sl]])
            plsc.addupdate_scatter(accv, [dstv[g, sl]], vals * ewv[g, sl])
            return c2
        return lax.fori_loop(0, CH // 16, ib, carry, unroll=8)
    lax.fori_loop(0, GPT, eb, 0)
    pltpu.sync_copy(accv, out_hbm.at[pl.ds(wid * NP, NP)])


def _tc_norm_body(degp_ref, dis_ref):
    deg = jnp.sum(degp_ref[...], axis=0, keepdims=True) + 1.0
    dis_ref[...] = lax.rsqrt(deg)


_tc_norm = pl.pallas_call(
    _tc_norm_body, out_shape=jax.ShapeDtypeStruct((1, NP), jnp.float32))


def _tc_in_body(x_ref, w_ref, dis_ref, out_ref):
    h = jnp.dot(x_ref[...], w_ref[...], preferred_element_type=jnp.float32)
    out_ref[...] = h * dis_ref[...]


_tc_in = pl.pallas_call(
    _tc_in_body, out_shape=jax.ShapeDtypeStruct((NP, FH), jnp.float32))


def _tc_mid_body(accp_ref, hp_ref, dis_ref, b_ref, w_ref, out_ref):
    acc = accp_ref[0] + accp_ref[1] + hp_ref[...]
    o = jnp.maximum(acc * dis_ref[...] + b_ref[...], 0.0)
    out_ref[...] = jnp.dot(
        o, w_ref[...], preferred_element_type=jnp.float32) * dis_ref[...]


def _tc_mid(accp, hp, dis_col, b, w):
    return pl.pallas_call(
        _tc_mid_body,
        out_shape=jax.ShapeDtypeStruct((NP, w.shape[1]), jnp.float32),
    )(accp, hp, dis_col, b, w)


def _tc_out_body(accp_ref, h3p_ref, dis_ref, b_ref, out_ref):
    acc = jnp.sum(accp_ref[...], axis=0, keepdims=True) + h3p_ref[...]
    out_ref[...] = acc * dis_ref[...] + b_ref[...]


_tc_out = pl.pallas_call(
    _tc_out_body, out_shape=jax.ShapeDtypeStruct((1, NP), jnp.float32))


def kernel(x, edge_index, edge_attr, W1, b1, W2, b2, W3, b3):
    src = edge_index[0].astype(jnp.int32)
    dst = edge_index[1].astype(jnp.int32)
    ew = edge_attr.astype(jnp.float32)
    pad = E_PAD - src.shape[0]
    src2 = jnp.concatenate([src, jnp.zeros((pad,), jnp.int32)]).reshape(NROWS, CH)
    dst2 = jnp.concatenate([dst, jnp.zeros((pad,), jnp.int32)]).reshape(NROWS, CH)
    ew2 = jnp.concatenate([ew, jnp.zeros((pad,), jnp.float32)]).reshape(NROWS, CH)
    x_pad = jnp.pad(x, ((0, NP - N), (0, 0)))

    degp = _sc_deg(dst2, ew2).reshape(NW, NP)
    dis_row = _tc_norm(degp)
    dis_col = dis_row.reshape(NP, 1)
    h1p = _tc_in(x_pad, W1, dis_col)
    acc1 = _sc_edge16(src2, dst2, ew2, h1p)
    h2p = _tc_mid(acc1, h1p, dis_col, b1.reshape(1, FH), W2)
    acc2 = _sc_edge16(src2, dst2, ew2, h2p)
    h3p = _tc_mid(acc2, h2p, dis_col, b2.reshape(1, FH), W3)
    acc3 = _sc_edge1(src2, dst2, ew2, h3p.reshape(NP)).reshape(NW, NP)
    out_row = _tc_out(acc3, h3p.reshape(1, NP), dis_row, b3.reshape(1, 1))
    return out_row.reshape(NP, 1)[:N]


# speedup vs baseline: 45.9520x; 1.3775x over previous
import functools

import jax
import jax.numpy as jnp
from jax import lax
from jax.experimental import pallas as pl
from jax.experimental.pallas import tpu as pltpu
from jax.experimental.pallas import tpu_sc as plsc

N = 10000
NP = 10240
FH = 16
NC = 2
NS = 16
NW = NC * NS
CH = 128
GPT = 80
EPT = CH * GPT
E_PAD = NW * EPT
NROWS = E_PAD // CH
N_PER_S = NP // NS
NBUF_G = 4
NBUF_S = 2

_mesh = plsc.VectorSubcoreMesh(
    core_axis_name="c", subcore_axis_name="s", num_cores=NC, num_subcores=NS)


def _worker_id():
    return lax.axis_index("s") * NC + lax.axis_index("c")


@functools.partial(
    pl.kernel,
    out_type=jax.ShapeDtypeStruct((NW * NP,), jnp.float32),
    mesh=_mesh,
    compiler_params=pltpu.CompilerParams(needs_layout_passes=False, use_tc_tiling_on_sc=False),
    scratch_types=[
        pltpu.VMEM((GPT, CH), jnp.int32),
        pltpu.VMEM((GPT, CH), jnp.float32),
        pltpu.VMEM((NP,), jnp.float32),
    ],
)
def _sc_deg(dst_hbm, ew_hbm, out_hbm, dstv, ewv, accv):
    wid = _worker_id()
    pltpu.sync_copy(dst_hbm.at[pl.ds(wid * GPT, GPT)], dstv)
    pltpu.sync_copy(ew_hbm.at[pl.ds(wid * GPT, GPT)], ewv)
    z = jnp.zeros((16,), jnp.float32)

    def zb(i, carry):
        accv[pl.ds(i * 16, 16)] = z
        return carry
    lax.fori_loop(0, NP // 16, zb, 0, unroll=8)

    def eb(g, carry):
        def ib(k, c2):
            idx = dstv[g, pl.ds(k * 16, 16)]
            w = ewv[g, pl.ds(k * 16, 16)]
            plsc.addupdate_scatter(accv, [idx], w)
            return c2
        return lax.fori_loop(0, CH // 16, ib, carry, unroll=8)
    lax.fori_loop(0, GPT, eb, 0)
    pltpu.sync_copy(accv, out_hbm.at[pl.ds(wid * NP, NP)])


@functools.partial(
    pl.kernel,
    out_type=jax.ShapeDtypeStruct((NC, NP, FH), jnp.float32),
    mesh=_mesh,
    compiler_params=pltpu.CompilerParams(needs_layout_passes=False, use_tc_tiling_on_sc=False),
    scratch_types=[
        pltpu.VMEM((GPT, CH), jnp.int32),
        pltpu.VMEM((GPT, CH), jnp.int32),
        pltpu.VMEM((GPT, CH), jnp.float32),
        pltpu.VMEM((NBUF_G, CH, FH), jnp.float32),
        pltpu.VMEM((NBUF_S, CH, FH), jnp.float32),
        pltpu.VMEM_SHARED((NP, FH), jnp.float32),
        pltpu.SemaphoreType.DMA((NBUF_G,)),
        pltpu.SemaphoreType.DMA((NBUF_S,)),
    ],
)
def _sc_edge16(src_hbm, dst_hbm, ew_hbm, hp_hbm, out_hbm,
               srcv, dstv, ewv, ga, sb, accs, gsem, ssem):
    s = lax.axis_index("s")
    c = lax.axis_index("c")
    wid = s * NC + c
    pltpu.sync_copy(src_hbm.at[pl.ds(wid * GPT, GPT)], srcv)
    pltpu.sync_copy(dst_hbm.at[pl.ds(wid * GPT, GPT)], dstv)
    pltpu.sync_copy(ew_hbm.at[pl.ds(wid * GPT, GPT)], ewv)

    z = jnp.zeros((16,), jnp.float32)

    def zb(i, carry):
        ga[0, i] = z
        return carry
    lax.fori_loop(0, CH, zb, 0, unroll=8)
    for q in range(N_PER_S // CH):
        pltpu.sync_copy(ga.at[0], accs.at[pl.ds(s * N_PER_S + q * CH, CH)])
    plsc.subcore_barrier()

    def start_gather(b, g):
        pltpu.async_copy(hp_hbm.at[srcv.at[g]], ga.at[b], gsem.at[b])

    def wait_gather(b, g):
        pltpu.make_async_copy(hp_hbm.at[srcv.at[g]], ga.at[b],
                              gsem.at[b]).wait()

    def start_scatter(v, g):
        pltpu.async_copy(sb.at[v], accs.at[dstv.at[g]], ssem.at[v],
                         add=True)

    def wait_scatter(v, g):
        pltpu.make_async_copy(sb.at[v], accs.at[dstv.at[g]],
                              ssem.at[v]).wait()

    for b in range(NBUF_G):
        start_gather(b, b)

    def gb(o, carry):
        for b in range(NBUF_G):
            g = o * NBUF_G + b
            v = b % NBUF_S
            wait_gather(b, g)

            if b >= NBUF_S:
                wait_scatter(v, g - NBUF_S)
            else:
                @pl.when(o > 0)
                def _():
                    wait_scatter(v, g - NBUF_S)

            def mb(j, c2):
                wv = ewv[g, pl.ds(j * 16, 16)]
                for t in range(16):
                    sb[v, j * 16 + t] = ga[b, j * 16 + t] * wv[t]
                return c2
            lax.fori_loop(0, CH // 16, mb, 0)
            start_scatter(v, g)

            @pl.when(o < GPT // NBUF_G - 1)
            def _():
                start_gather(b, g + NBUF_G)
        return carry
    lax.fori_loop(0, GPT // NBUF_G, gb, 0)
    for v in range(NBUF_S):
        wait_scatter(v, GPT - NBUF_S + v)

    plsc.subcore_barrier()
    pltpu.sync_copy(accs.at[pl.ds(s * N_PER_S, N_PER_S)],
                    out_hbm.at[c, pl.ds(s * N_PER_S, N_PER_S)])


@functools.partial(
    pl.kernel,
    out_type=jax.ShapeDtypeStruct((NW * NP,), jnp.float32),
    mesh=_mesh,
    compiler_params=pltpu.CompilerParams(needs_layout_passes=False, use_tc_tiling_on_sc=False),
    scratch_types=[
        pltpu.VMEM((GPT, CH), jnp.int32),
        pltpu.VMEM((GPT, CH), jnp.int32),
        pltpu.VMEM((GPT, CH), jnp.float32),
        pltpu.VMEM((NP,), jnp.float32),
        pltpu.VMEM((NP,), jnp.float32),
    ],
)
def _sc_edge1(src_hbm, dst_hbm, ew_hbm, h3_hbm, out_hbm,
              srcv, dstv, ewv, hv, accv):
    wid = _worker_id()
    pltpu.sync_copy(src_hbm.at[pl.ds(wid * GPT, GPT)], srcv)
    pltpu.sync_copy(dst_hbm.at[pl.ds(wid * GPT, GPT)], dstv)
    pltpu.sync_copy(ew_hbm.at[pl.ds(wid * GPT, GPT)], ewv)
    pltpu.sync_copy(h3_hbm, hv)
    z = jnp.zeros((16,), jnp.float32)

    def zb(i, carry):
        accv[pl.ds(i * 16, 16)] = z
        return carry
    lax.fori_loop(0, NP // 16, zb, 0, unroll=8)

    def eb(g, carry):
        def ib(k, c2):
            sl = pl.ds(k * 16, 16)
            vals = plsc.load_gather(hv, [srcv[g, sl]])
            plsc.addupdate_scatter(accv, [dstv[g, sl]], vals * ewv[g, sl])
            return c2
        return lax.fori_loop(0, CH // 16, ib, carry, unroll=8)
    lax.fori_loop(0, GPT, eb, 0)
    pltpu.sync_copy(accv, out_hbm.at[pl.ds(wid * NP, NP)])


def _tc_norm_body(degp_ref, dis_ref):
    deg = jnp.sum(degp_ref[...], axis=0, keepdims=True) + 1.0
    dis_ref[...] = lax.rsqrt(deg)


_tc_norm = pl.pallas_call(
    _tc_norm_body, out_shape=jax.ShapeDtypeStruct((1, NP), jnp.float32))


def _tc_in_body(x_ref, w_ref, dis_ref, out_ref):
    h = jnp.dot(x_ref[...], w_ref[...], preferred_element_type=jnp.float32)
    out_ref[...] = h * dis_ref[...]


_tc_in = pl.pallas_call(
    _tc_in_body, out_shape=jax.ShapeDtypeStruct((NP, FH), jnp.float32))


def _tc_mid_body(accp_ref, hp_ref, dis_ref, b_ref, w_ref, out_ref):
    acc = accp_ref[0] + accp_ref[1] + hp_ref[...]
    o = jnp.maximum(acc * dis_ref[...] + b_ref[...], 0.0)
    out_ref[...] = jnp.dot(
        o, w_ref[...], preferred_element_type=jnp.float32) * dis_ref[...]


def _tc_mid(accp, hp, dis_col, b, w):
    return pl.pallas_call(
        _tc_mid_body,
        out_shape=jax.ShapeDtypeStruct((NP, w.shape[1]), jnp.float32),
    )(accp, hp, dis_col, b, w)


def _tc_out_body(accp_ref, h3p_ref, dis_ref, b_ref, out_ref):
    acc = jnp.sum(accp_ref[...], axis=0, keepdims=True) + h3p_ref[...]
    out_ref[...] = acc * dis_ref[...] + b_ref[...]


_tc_out = pl.pallas_call(
    _tc_out_body, out_shape=jax.ShapeDtypeStruct((1, NP), jnp.float32))


def kernel(x, edge_index, edge_attr, W1, b1, W2, b2, W3, b3):
    src = edge_index[0].astype(jnp.int32)
    dst = edge_index[1].astype(jnp.int32)
    ew = edge_attr.astype(jnp.float32)
    pad = E_PAD - src.shape[0]
    src2 = jnp.concatenate([src, jnp.zeros((pad,), jnp.int32)]).reshape(NROWS, CH)
    dst2 = jnp.concatenate([dst, jnp.zeros((pad,), jnp.int32)]).reshape(NROWS, CH)
    ew2 = jnp.concatenate([ew, jnp.zeros((pad,), jnp.float32)]).reshape(NROWS, CH)
    x_pad = jnp.pad(x, ((0, NP - N), (0, 0)))

    degp = _sc_deg(dst2, ew2).reshape(NW, NP)
    dis_row = _tc_norm(degp)
    dis_col = dis_row.reshape(NP, 1)
    h1p = _tc_in(x_pad, W1, dis_col)
    acc1 = _sc_edge16(src2, dst2, ew2, h1p)
    h2p = _tc_mid(acc1, h1p, dis_col, b1.reshape(1, FH), W2)
    acc2 = _sc_edge16(src2, dst2, ew2, h2p)
    h3p = _tc_mid(acc2, h2p, dis_col, b2.reshape(1, FH), W3)
    acc3 = _sc_edge1(src2, dst2, ew2, h3p.reshape(NP)).reshape(NW, NP)
    out_row = _tc_out(acc3, h3p.reshape(1, NP), dis_row, b3.reshape(1, 1))
    return out_row.reshape(NP, 1)[:N]


# speedup vs baseline: 61.3434x; 1.3349x over previous
import functools

import jax
import jax.numpy as jnp
from jax import lax
from jax.experimental import pallas as pl
from jax.experimental.pallas import tpu as pltpu
from jax.experimental.pallas import tpu_sc as plsc

N = 10000
NP = 10240
FH = 16
NC = 2
NS = 16
NW = NC * NS
CH = 128
GPT = 80
EPT = CH * GPT
E_PAD = NW * EPT
NROWS = E_PAD // CH
N_PER_S = NP // NS
NBUF_G = 4
NBUF_S = 2

_mesh = plsc.VectorSubcoreMesh(
    core_axis_name="c", subcore_axis_name="s", num_cores=NC, num_subcores=NS)

_sc_params = pltpu.CompilerParams(
    needs_layout_passes=False, use_tc_tiling_on_sc=False)


def _worker_id():
    return lax.axis_index("s") * NC + lax.axis_index("c")


@functools.partial(
    pl.kernel,
    out_type=jax.ShapeDtypeStruct((NW * N,), jnp.float32),
    mesh=_mesh,
    compiler_params=_sc_params,
    scratch_types=[
        pltpu.VMEM((GPT, CH), jnp.int32),
        pltpu.VMEM((GPT, CH), jnp.float32),
        pltpu.VMEM((N,), jnp.float32),
    ],
)
def _sc_deg(dst_hbm, ew_hbm, out_hbm, dstv, ewv, accv):
    wid = _worker_id()
    pltpu.sync_copy(dst_hbm.at[pl.ds(wid * GPT, GPT)], dstv)
    pltpu.sync_copy(ew_hbm.at[pl.ds(wid * GPT, GPT)], ewv)
    z = jnp.zeros((16,), jnp.float32)

    def zb(i, carry):
        accv[pl.ds(i * 16, 16)] = z
        return carry
    lax.fori_loop(0, N // 16, zb, 0, unroll=8)

    def eb(g, carry):
        def ib(k, c2):
            idx = dstv[g, pl.ds(k * 16, 16)]
            w = ewv[g, pl.ds(k * 16, 16)]
            plsc.addupdate_scatter(accv, [idx], w)
            return c2
        return lax.fori_loop(0, CH // 16, ib, carry, unroll=8)
    lax.fori_loop(0, GPT, eb, 0)
    pltpu.sync_copy(accv, out_hbm.at[pl.ds(wid * N, N)])


@functools.partial(
    pl.kernel,
    out_type=jax.ShapeDtypeStruct((NC, NP, FH), jnp.float32),
    mesh=_mesh,
    compiler_params=_sc_params,
    scratch_types=[
        pltpu.VMEM((GPT, CH), jnp.int32),
        pltpu.VMEM((GPT, CH), jnp.int32),
        pltpu.VMEM((GPT, CH), jnp.float32),
        pltpu.VMEM((NBUF_G, CH, FH), jnp.float32),
        pltpu.VMEM((NBUF_S, CH, FH), jnp.float32),
        pltpu.VMEM_SHARED((NP, FH), jnp.float32),
        pltpu.VMEM_SHARED((N, FH), jnp.float32),
        pltpu.SemaphoreType.DMA((NBUF_G,)),
        pltpu.SemaphoreType.DMA((NBUF_S,)),
    ],
)
def _sc_edge16(src_hbm, dst_hbm, ew_hbm, hp_hbm, out_hbm,
               srcv, dstv, ewv, ga, sb, accs, hps, gsem, ssem):
    s = lax.axis_index("s")
    c = lax.axis_index("c")
    wid = s * NC + c
    pltpu.sync_copy(src_hbm.at[pl.ds(wid * GPT, GPT)], srcv)
    pltpu.sync_copy(dst_hbm.at[pl.ds(wid * GPT, GPT)], dstv)
    pltpu.sync_copy(ew_hbm.at[pl.ds(wid * GPT, GPT)], ewv)

    nh = N // NS
    pltpu.sync_copy(hp_hbm.at[pl.ds(s * nh, nh)], hps.at[pl.ds(s * nh, nh)])

    z = jnp.zeros((16,), jnp.float32)

    def zb(i, carry):
        ga[0, i] = z
        return carry
    lax.fori_loop(0, CH, zb, 0, unroll=8)
    for q in range(N_PER_S // CH):
        pltpu.sync_copy(ga.at[0], accs.at[pl.ds(s * N_PER_S + q * CH, CH)])
    plsc.subcore_barrier()

    def start_gather(b, g):
        pltpu.async_copy(hps.at[srcv.at[g]], ga.at[b], gsem.at[b])

    def wait_gather(b, g):
        pltpu.make_async_copy(hps.at[srcv.at[g]], ga.at[b],
                              gsem.at[b]).wait()

    def start_scatter(v, g):
        pltpu.async_copy(sb.at[v], accs.at[dstv.at[g]], ssem.at[v],
                         add=True)

    def wait_scatter(v, g):
        pltpu.make_async_copy(sb.at[v], accs.at[dstv.at[g]],
                              ssem.at[v]).wait()

    for b in range(NBUF_G):
        start_gather(b, b)

    def gb(o, carry):
        for b in range(NBUF_G):
            g = o * NBUF_G + b
            v = b % NBUF_S
            wait_gather(b, g)

            if b >= NBUF_S:
                wait_scatter(v, g - NBUF_S)
            else:
                @pl.when(o > 0)
                def _():
                    wait_scatter(v, g - NBUF_S)

            def mb(j, c2):
                wv = ewv[g, pl.ds(j * 16, 16)]
                for t in range(16):
                    sb[v, j * 16 + t] = ga[b, j * 16 + t] * wv[t]
                return c2
            lax.fori_loop(0, CH // 16, mb, 0)
            start_scatter(v, g)

            @pl.when(o < GPT // NBUF_G - 1)
            def _():
                start_gather(b, g + NBUF_G)
        return carry
    lax.fori_loop(0, GPT // NBUF_G, gb, 0)
    for v in range(NBUF_S):
        wait_scatter(v, GPT - NBUF_S + v)

    plsc.subcore_barrier()
    pltpu.sync_copy(accs.at[pl.ds(s * N_PER_S, N_PER_S)],
                    out_hbm.at[c, pl.ds(s * N_PER_S, N_PER_S)])


@functools.partial(
    pl.kernel,
    out_type=jax.ShapeDtypeStruct((NW * N,), jnp.float32),
    mesh=_mesh,
    compiler_params=_sc_params,
    scratch_types=[
        pltpu.VMEM((GPT, CH), jnp.int32),
        pltpu.VMEM((GPT, CH), jnp.int32),
        pltpu.VMEM((GPT, CH), jnp.float32),
        pltpu.VMEM((N,), jnp.float32),
        pltpu.VMEM((N,), jnp.float32),
    ],
)
def _sc_edge1(src_hbm, dst_hbm, ew_hbm, h3_hbm, out_hbm,
              srcv, dstv, ewv, hv, accv):
    wid = _worker_id()
    pltpu.sync_copy(src_hbm.at[pl.ds(wid * GPT, GPT)], srcv)
    pltpu.sync_copy(dst_hbm.at[pl.ds(wid * GPT, GPT)], dstv)
    pltpu.sync_copy(ew_hbm.at[pl.ds(wid * GPT, GPT)], ewv)
    pltpu.sync_copy(h3_hbm, hv)
    z = jnp.zeros((16,), jnp.float32)

    def zb(i, carry):
        accv[pl.ds(i * 16, 16)] = z
        return carry
    lax.fori_loop(0, N // 16, zb, 0, unroll=8)

    def eb(g, carry):
        def ib(k, c2):
            sl = pl.ds(k * 16, 16)
            vals = plsc.load_gather(hv, [srcv[g, sl]])
            plsc.addupdate_scatter(accv, [dstv[g, sl]], vals * ewv[g, sl])
            return c2
        return lax.fori_loop(0, CH // 16, ib, carry, unroll=8)
    lax.fori_loop(0, GPT, eb, 0)
    pltpu.sync_copy(accv, out_hbm.at[pl.ds(wid * N, N)])


def _tc_first_body(degp_ref, x_ref, w_ref, dis_ref, hp_ref):
    deg = jnp.sum(degp_ref[...], axis=0, keepdims=True) + 1.0
    dis_col = jnp.reshape(lax.rsqrt(deg), (N, 1))
    dis_ref[...] = dis_col
    h = jnp.dot(x_ref[...], w_ref[...], preferred_element_type=jnp.float32)
    hp_ref[...] = h * dis_col


_tc_first = pl.pallas_call(
    _tc_first_body,
    out_shape=(jax.ShapeDtypeStruct((N, 1), jnp.float32),
               jax.ShapeDtypeStruct((N, FH), jnp.float32)))


def _tc_mid_body(accp_ref, hp_ref, dis_ref, b_ref, w_ref, out_ref):
    acc = accp_ref[0, :N] + accp_ref[1, :N] + hp_ref[...]
    o = jnp.maximum(acc * dis_ref[...] + b_ref[...], 0.0)
    out_ref[...] = jnp.dot(
        o, w_ref[...], preferred_element_type=jnp.float32) * dis_ref[...]


def _tc_mid(accp, hp, dis_col, b, w):
    return pl.pallas_call(
        _tc_mid_body,
        out_shape=jax.ShapeDtypeStruct((N, w.shape[1]), jnp.float32),
    )(accp, hp, dis_col, b, w)


def _tc_out_body(accp_ref, h3p_ref, dis_ref, b_ref, out_ref):
    acc = jnp.reshape(jnp.sum(accp_ref[...], axis=0), (N, 1))
    out_ref[...] = (acc + h3p_ref[...]) * dis_ref[...] + b_ref[...]


_tc_out = pl.pallas_call(
    _tc_out_body, out_shape=jax.ShapeDtypeStruct((N, 1), jnp.float32))


def kernel(x, edge_index, edge_attr, W1, b1, W2, b2, W3, b3):
    src = edge_index[0].astype(jnp.int32)
    dst = edge_index[1].astype(jnp.int32)
    ew = edge_attr.astype(jnp.float32)
    pad = E_PAD - src.shape[0]
    src2 = jnp.concatenate([src, jnp.zeros((pad,), jnp.int32)]).reshape(NROWS, CH)
    dst2 = jnp.concatenate([dst, jnp.zeros((pad,), jnp.int32)]).reshape(NROWS, CH)
    ew2 = jnp.concatenate([ew, jnp.zeros((pad,), jnp.float32)]).reshape(NROWS, CH)

    degp = _sc_deg(dst2, ew2).reshape(NW, N)
    dis_col, h1p = _tc_first(degp, x, W1)
    acc1 = _sc_edge16(src2, dst2, ew2, h1p)
    h2p = _tc_mid(acc1, h1p, dis_col, b1.reshape(1, FH), W2)
    acc2 = _sc_edge16(src2, dst2, ew2, h2p)
    h3p = _tc_mid(acc2, h2p, dis_col, b2.reshape(1, FH), W3)
    acc3 = _sc_edge1(src2, dst2, ew2, h3p.reshape(N)).reshape(NW, N)
    return _tc_out(acc3, h3p, dis_col, b3.reshape(1, 1))


# speedup vs baseline: 72.2948x; 1.1785x over previous
import functools

import jax
import jax.numpy as jnp
from jax import lax
from jax.experimental import pallas as pl
from jax.experimental.pallas import tpu as pltpu
from jax.experimental.pallas import tpu_sc as plsc

N = 10000
NP = 10240
FH = 16
NC = 2
NS = 16
NW = NC * NS
E = 320000
CH = 128
ER = E // CH
RPW = ER // NW
NEXTRA = ER % NW
N_PER_S = NP // NS
HR = N * FH // CH
HRS = HR // NS
NBUF = 6

_mesh = plsc.VectorSubcoreMesh(
    core_axis_name="c", subcore_axis_name="s", num_cores=NC, num_subcores=NS)

_sc_params = pltpu.CompilerParams(
    needs_layout_passes=False, use_tc_tiling_on_sc=False)


def _worker_id():
    return lax.axis_index("s") * NC + lax.axis_index("c")


def _stage_edges(ei_hbm, ew_hbm, wid, dstv, ewv, srcv=None):
    if srcv is not None:
        pltpu.sync_copy(ei_hbm.at[pl.ds(wid * RPW, RPW)],
                        srcv.at[pl.ds(0, RPW)])
    pltpu.sync_copy(ei_hbm.at[pl.ds(ER + wid * RPW, RPW)],
                    dstv.at[pl.ds(0, RPW)])
    pltpu.sync_copy(ew_hbm.at[pl.ds(wid * RPW, RPW)], ewv.at[pl.ds(0, RPW)])

    @pl.when(wid < NEXTRA)
    def _():
        if srcv is not None:
            pltpu.sync_copy(ei_hbm.at[pl.ds(NW * RPW + wid, 1)],
                            srcv.at[pl.ds(RPW, 1)])
        pltpu.sync_copy(ei_hbm.at[pl.ds(ER + NW * RPW + wid, 1)],
                        dstv.at[pl.ds(RPW, 1)])
        pltpu.sync_copy(ew_hbm.at[pl.ds(NW * RPW + wid, 1)],
                        ewv.at[pl.ds(RPW, 1)])


@functools.partial(
    pl.kernel,
    out_type=jax.ShapeDtypeStruct((NW * N,), jnp.float32),
    mesh=_mesh,
    compiler_params=_sc_params,
    scratch_types=[
        pltpu.VMEM((RPW + 1, CH), jnp.int32),
        pltpu.VMEM((RPW + 1, CH), jnp.float32),
        pltpu.VMEM((N,), jnp.float32),
    ],
)
def _sc_deg(ei_hbm, ew_hbm, out_hbm, dstv, ewv, accv):
    wid = _worker_id()
    _stage_edges(ei_hbm, ew_hbm, wid, dstv, ewv)
    z = jnp.zeros((16,), jnp.float32)

    def zb(i, carry):
        accv[pl.ds(i * 16, 16)] = z
        return carry
    lax.fori_loop(0, N // 16, zb, 0, unroll=8)

    def eb(g, carry):
        def ib(k, c2):
            idx = dstv[g, pl.ds(k * 16, 16)]
            w = ewv[g, pl.ds(k * 16, 16)]
            plsc.addupdate_scatter(accv, [idx], w)
            return c2
        return lax.fori_loop(0, CH // 16, ib, carry, unroll=8)
    lax.fori_loop(0, RPW, eb, 0)

    @pl.when(wid < NEXTRA)
    def _():
        eb(RPW, 0)
    pltpu.sync_copy(accv, out_hbm.at[pl.ds(wid * N, N)])


@functools.partial(
    pl.kernel,
    out_type=jax.ShapeDtypeStruct((NC, NP, FH), jnp.float32),
    mesh=_mesh,
    compiler_params=_sc_params,
    scratch_types=[
        pltpu.VMEM((RPW + 1, CH), jnp.int32),
        pltpu.VMEM((RPW + 1, CH), jnp.int32),
        pltpu.VMEM((RPW + 1, CH), jnp.float32),
        pltpu.VMEM((NBUF, CH, FH), jnp.float32),
        pltpu.VMEM((NBUF, CH, FH), jnp.float32),
        pltpu.VMEM_SHARED((NP, FH), jnp.float32),
        pltpu.VMEM_SHARED((N, FH), jnp.float32),
        pltpu.SemaphoreType.DMA((NBUF,)),
        pltpu.SemaphoreType.DMA((NBUF,)),
    ],
)
def _sc_edge16(ei_hbm, ew_hbm, hp_hbm, out_hbm,
               srcv, dstv, ewv, ga, sb, accs, hps, gsem, ssem):
    s = lax.axis_index("s")
    c = lax.axis_index("c")
    wid = s * NC + c
    _stage_edges(ei_hbm, ew_hbm, wid, dstv, ewv, srcv)

    nh = N // NS
    pltpu.sync_copy(hp_hbm.at[pl.ds(s * nh, nh)], hps.at[pl.ds(s * nh, nh)])

    z = jnp.zeros((16,), jnp.float32)

    def zb(i, carry):
        ga[0, i] = z
        return carry
    lax.fori_loop(0, CH, zb, 0, unroll=8)
    for q in range(N_PER_S // CH):
        pltpu.sync_copy(ga.at[0], accs.at[pl.ds(s * N_PER_S + q * CH, CH)])
    plsc.subcore_barrier()

    def start_gather(b, g):
        pltpu.async_copy(hps.at[srcv.at[g]], ga.at[b], gsem.at[b])

    def wait_gather(b, g):
        pltpu.make_async_copy(hps.at[srcv.at[g]], ga.at[b],
                              gsem.at[b]).wait()

    def start_scatter(b, g):
        pltpu.async_copy(sb.at[b], accs.at[dstv.at[g]], ssem.at[b],
                         add=True)

    def wait_scatter(b, g):
        pltpu.make_async_copy(sb.at[b], accs.at[dstv.at[g]],
                              ssem.at[b]).wait()

    def scale_group(b, g):
        def mb(j, c2):
            wv = ewv[g, pl.ds(j * 16, 16)]
            for t in range(16):
                sb[b, j * 16 + t] = ga[b, j * 16 + t] * wv[t]
            return c2
        lax.fori_loop(0, CH // 16, mb, 0)

    for b in range(NBUF):
        start_gather(b, b)

    def gb(o, carry):
        for b in range(NBUF):
            g = o * NBUF + b
            wait_gather(b, g)

            @pl.when(o > 0)
            def _():
                wait_scatter(b, g - NBUF)
            scale_group(b, g)
            start_scatter(b, g)

            @pl.when(o < RPW // NBUF - 1)
            def _():
                start_gather(b, g + NBUF)
        return carry
    lax.fori_loop(0, RPW // NBUF, gb, 0)
    for b in range(NBUF):
        wait_scatter(b, RPW - NBUF + b)

    @pl.when(wid < NEXTRA)
    def _():
        pltpu.async_copy(hps.at[srcv.at[RPW]], ga.at[0], gsem.at[0]).wait()
        scale_group(0, RPW)
        pltpu.async_copy(sb.at[0], accs.at[dstv.at[RPW]], ssem.at[0],
                         add=True).wait()

    plsc.subcore_barrier()
    pltpu.sync_copy(accs.at[pl.ds(s * N_PER_S, N_PER_S)],
                    out_hbm.at[c, pl.ds(s * N_PER_S, N_PER_S)])


@functools.partial(
    pl.kernel,
    out_type=jax.ShapeDtypeStruct((NW * N,), jnp.float32),
    mesh=_mesh,
    compiler_params=_sc_params,
    scratch_types=[
        pltpu.VMEM((RPW + 1, CH), jnp.int32),
        pltpu.VMEM((RPW + 1, CH), jnp.int32),
        pltpu.VMEM((RPW + 1, CH), jnp.float32),
        pltpu.VMEM((N,), jnp.float32),
        pltpu.VMEM((N,), jnp.float32),
    ],
)
def _sc_edge1(ei_hbm, ew_hbm, h3_hbm, out_hbm,
              srcv, dstv, ewv, hv, accv):
    wid = _worker_id()
    _stage_edges(ei_hbm, ew_hbm, wid, dstv, ewv, srcv)
    pltpu.sync_copy(h3_hbm, hv)
    z = jnp.zeros((16,), jnp.float32)

    def zb(i, carry):
        accv[pl.ds(i * 16, 16)] = z
        return carry
    lax.fori_loop(0, N // 16, zb, 0, unroll=8)

    def eb(g, carry):
        def ib(k, c2):
            sl = pl.ds(k * 16, 16)
            vals = plsc.load_gather(hv, [srcv[g, sl]])
            plsc.addupdate_scatter(accv, [dstv[g, sl]], vals * ewv[g, sl])
            return c2
        return lax.fori_loop(0, CH // 16, ib, carry, unroll=8)
    lax.fori_loop(0, RPW, eb, 0)

    @pl.when(wid < NEXTRA)
    def _():
        eb(RPW, 0)
    pltpu.sync_copy(accv, out_hbm.at[pl.ds(wid * N, N)])


def _tc_first_body(degp_ref, x_ref, w_ref, dis_ref, hp_ref):
    deg = jnp.sum(degp_ref[...], axis=0, keepdims=True) + 1.0
    dis_col = jnp.reshape(lax.rsqrt(deg), (N, 1))
    dis_ref[...] = dis_col
    h = jnp.dot(x_ref[...], w_ref[...], preferred_element_type=jnp.float32)
    hp_ref[...] = h * dis_col


_tc_first = pl.pallas_call(
    _tc_first_body,
    out_shape=(jax.ShapeDtypeStruct((N, 1), jnp.float32),
               jax.ShapeDtypeStruct((N, FH), jnp.float32)))


def _relu_comb(accp_ref, hp_ref, dis_ref, b_ref):
    acc = accp_ref[0, :N] + accp_ref[1, :N] + hp_ref[...]
    return jnp.maximum(acc * dis_ref[...] + b_ref[...], 0.0)


def _tc_mid_body(accp_ref, hp_ref, dis_ref, b_ref, w_ref, out_ref):
    o = _relu_comb(accp_ref, hp_ref, dis_ref, b_ref)
    out_ref[...] = jnp.dot(
        o, w_ref[...], preferred_element_type=jnp.float32) * dis_ref[...]


_tc_mid = pl.pallas_call(
    _tc_mid_body, out_shape=jax.ShapeDtypeStruct((N, FH), jnp.float32))


def _tc_mid3_body(accp_ref, hp_ref, dis_ref, b_ref, w_ref, out_ref):
    o = _relu_comb(accp_ref, hp_ref, dis_ref, b_ref)
    out_ref[...] = jnp.dot(
        o, w_ref[...], preferred_element_type=jnp.float32) * dis_ref[...]


_tc_mid3 = pl.pallas_call(
    _tc_mid3_body, out_shape=jax.ShapeDtypeStruct((N, 1), jnp.float32))


def _tc_out_body(accp_ref, h3p_ref, dis_ref, b_ref, out_ref):
    acc = jnp.reshape(jnp.sum(accp_ref[...], axis=0), (N, 1))
    out_ref[...] = (acc + h3p_ref[...]) * dis_ref[...] + b_ref[...]


_tc_out = pl.pallas_call(
    _tc_out_body, out_shape=jax.ShapeDtypeStruct((N, 1), jnp.float32))


def kernel(x, edge_index, edge_attr, W1, b1, W2, b2, W3, b3):
    ei = edge_index.astype(jnp.int32).reshape(2 * ER, CH)
    ew = edge_attr.astype(jnp.float32).reshape(ER, CH)
    degp = _sc_deg(ei, ew).reshape(NW, N)
    dis_col, h1p = _tc_first(degp, x, W1)
    acc1 = _sc_edge16(ei, ew, h1p)
    h2p = _tc_mid(acc1, h1p, dis_col, b1.reshape(1, FH), W2)
    acc2 = _sc_edge16(ei, ew, h2p)
    h3p = _tc_mid3(acc2, h2p, dis_col, b2.reshape(1, FH), W3)
    acc3 = _sc_edge1(ei, ew, h3p.reshape(N)).reshape(NW, N)
    return _tc_out(acc3, h3p, dis_col, b3.reshape(1, 1))


# speedup vs baseline: 73.1991x; 1.0125x over previous
import functools

import jax
import jax.numpy as jnp
from jax import lax
from jax.experimental import pallas as pl
from jax.experimental.pallas import tpu as pltpu
from jax.experimental.pallas import tpu_sc as plsc

N = 10000
NP = 10240
FH = 16
NC = 2
NS = 16
NW = NC * NS
E = 320000
CH = 128
ER = E // CH
RPW = ER // NW
NEXTRA = ER % NW
N_PER_S = NP // NS
HR = N * FH // CH
HRS = HR // NS
NBUF = 6

_mesh = plsc.VectorSubcoreMesh(
    core_axis_name="c", subcore_axis_name="s", num_cores=NC, num_subcores=NS)

_sc_params = pltpu.CompilerParams(
    needs_layout_passes=False, use_tc_tiling_on_sc=False)


def _worker_id():
    return lax.axis_index("s") * NC + lax.axis_index("c")


def _stage_edges(ei_hbm, ew_hbm, wid, dstv, ewv, srcv=None):
    if srcv is not None:
        pltpu.sync_copy(ei_hbm.at[pl.ds(wid * RPW, RPW)],
                        srcv.at[pl.ds(0, RPW)])
    pltpu.sync_copy(ei_hbm.at[pl.ds(ER + wid * RPW, RPW)],
                    dstv.at[pl.ds(0, RPW)])
    pltpu.sync_copy(ew_hbm.at[pl.ds(wid * RPW, RPW)], ewv.at[pl.ds(0, RPW)])

    @pl.when(wid < NEXTRA)
    def _():
        if srcv is not None:
            pltpu.sync_copy(ei_hbm.at[pl.ds(NW * RPW + wid, 1)],
                            srcv.at[pl.ds(RPW, 1)])
        pltpu.sync_copy(ei_hbm.at[pl.ds(ER + NW * RPW + wid, 1)],
                        dstv.at[pl.ds(RPW, 1)])
        pltpu.sync_copy(ew_hbm.at[pl.ds(NW * RPW + wid, 1)],
                        ewv.at[pl.ds(RPW, 1)])


@functools.partial(
    pl.kernel,
    out_type=jax.ShapeDtypeStruct((NW * N,), jnp.float32),
    mesh=_mesh,
    compiler_params=_sc_params,
    scratch_types=[
        pltpu.VMEM((RPW + 1, CH), jnp.int32),
        pltpu.VMEM((RPW + 1, CH), jnp.float32),
        pltpu.VMEM((N,), jnp.float32),
    ],
)
def _sc_deg(ei_hbm, ew_hbm, out_hbm, dstv, ewv, accv):
    wid = _worker_id()
    _stage_edges(ei_hbm, ew_hbm, wid, dstv, ewv)
    z = jnp.zeros((16,), jnp.float32)

    def zb(i, carry):
        accv[pl.ds(i * 16, 16)] = z
        return carry
    lax.fori_loop(0, N // 16, zb, 0, unroll=8)

    def eb(g, carry):
        def ib(k, c2):
            idx = dstv[g, pl.ds(k * 16, 16)]
            w = ewv[g, pl.ds(k * 16, 16)]
            plsc.addupdate_scatter(accv, [idx], w)
            return c2
        return lax.fori_loop(0, CH // 16, ib, carry, unroll=8)
    lax.fori_loop(0, RPW, eb, 0)

    @pl.when(wid < NEXTRA)
    def _():
        eb(RPW, 0)
    pltpu.sync_copy(accv, out_hbm.at[pl.ds(wid * N, N)])


@functools.partial(
    pl.kernel,
    out_type=jax.ShapeDtypeStruct((NC, NP, FH), jnp.float32),
    mesh=_mesh,
    compiler_params=_sc_params,
    scratch_types=[
        pltpu.VMEM((RPW + 1, CH), jnp.int32),
        pltpu.VMEM((RPW + 1, CH), jnp.int32),
        pltpu.VMEM((RPW + 1, CH), jnp.float32),
        pltpu.VMEM((NBUF, CH, FH), jnp.float32),
        pltpu.VMEM((NBUF, CH, FH), jnp.float32),
        pltpu.VMEM_SHARED((NP, FH), jnp.float32),
        pltpu.VMEM_SHARED((N, FH), jnp.float32),
        pltpu.SemaphoreType.DMA((NBUF,)),
        pltpu.SemaphoreType.DMA((NBUF,)),
    ],
)
def _sc_edge16(ei_hbm, ew_hbm, hp_hbm, out_hbm,
               srcv, dstv, ewv, ga, sb, accs, hps, gsem, ssem):
    s = lax.axis_index("s")
    c = lax.axis_index("c")
    wid = s * NC + c
    _stage_edges(ei_hbm, ew_hbm, wid, dstv, ewv, srcv)

    nh = N // NS
    pltpu.sync_copy(hp_hbm.at[pl.ds(s * nh, nh)], hps.at[pl.ds(s * nh, nh)])

    z = jnp.zeros((16,), jnp.float32)

    def zb(i, carry):
        ga[0, i] = z
        return carry
    lax.fori_loop(0, CH, zb, 0, unroll=8)
    for q in range(N_PER_S // CH):
        pltpu.sync_copy(ga.at[0], accs.at[pl.ds(s * N_PER_S + q * CH, CH)])
    plsc.subcore_barrier()

    def start_gather(b, g):
        pltpu.async_copy(hps.at[srcv.at[g]], ga.at[b], gsem.at[b])

    def wait_gather(b, g):
        pltpu.make_async_copy(hps.at[srcv.at[g]], ga.at[b],
                              gsem.at[b]).wait()

    def start_scatter(b, g):
        pltpu.async_copy(sb.at[b], accs.at[dstv.at[g]], ssem.at[b],
                         add=True)

    def wait_scatter(b, g):
        pltpu.make_async_copy(sb.at[b], accs.at[dstv.at[g]],
                              ssem.at[b]).wait()

    def scale_group(b, g):
        def mb(j, c2):
            wv = ewv[g, pl.ds(j * 16, 16)]
            for t in range(16):
                sb[b, j * 16 + t] = ga[b, j * 16 + t] * wv[t]
            return c2
        lax.fori_loop(0, CH // 16, mb, 0, unroll=4)

    for b in range(NBUF):
        start_gather(b, b)

    def gb(o, carry):
        for b in range(NBUF):
            g = o * NBUF + b
            wait_gather(b, g)

            @pl.when(o > 0)
            def _():
                wait_scatter(b, g - NBUF)
            scale_group(b, g)
            start_scatter(b, g)

            @pl.when(o < RPW // NBUF - 1)
            def _():
                start_gather(b, g + NBUF)
        return carry
    lax.fori_loop(0, RPW // NBUF, gb, 0)
    for b in range(NBUF):
        wait_scatter(b, RPW - NBUF + b)

    @pl.when(wid < NEXTRA)
    def _():
        pltpu.async_copy(hps.at[srcv.at[RPW]], ga.at[0], gsem.at[0]).wait()
        scale_group(0, RPW)
        pltpu.async_copy(sb.at[0], accs.at[dstv.at[RPW]], ssem.at[0],
                         add=True).wait()

    plsc.subcore_barrier()
    pltpu.sync_copy(accs.at[pl.ds(s * N_PER_S, N_PER_S)],
                    out_hbm.at[c, pl.ds(s * N_PER_S, N_PER_S)])


@functools.partial(
    pl.kernel,
    out_type=jax.ShapeDtypeStruct((NW * N,), jnp.float32),
    mesh=_mesh,
    compiler_params=_sc_params,
    scratch_types=[
        pltpu.VMEM((RPW + 1, CH), jnp.int32),
        pltpu.VMEM((RPW + 1, CH), jnp.int32),
        pltpu.VMEM((RPW + 1, CH), jnp.float32),
        pltpu.VMEM((NP,), jnp.float32),
        pltpu.VMEM((N,), jnp.float32),
    ],
)
def _sc_edge1(ei_hbm, ew_hbm, h3_hbm, out_hbm,
              srcv, dstv, ewv, hv, accv):
    wid = _worker_id()
    _stage_edges(ei_hbm, ew_hbm, wid, dstv, ewv, srcv)
    pltpu.sync_copy(h3_hbm, hv)
    z = jnp.zeros((16,), jnp.float32)

    def zb(i, carry):
        accv[pl.ds(i * 16, 16)] = z
        return carry
    lax.fori_loop(0, N // 16, zb, 0, unroll=8)

    def eb(g, carry):
        def ib(k, c2):
            sl = pl.ds(k * 16, 16)
            vals = plsc.load_gather(hv, [srcv[g, sl]])
            plsc.addupdate_scatter(accv, [dstv[g, sl]], vals * ewv[g, sl])
            return c2
        return lax.fori_loop(0, CH // 16, ib, carry, unroll=8)
    lax.fori_loop(0, RPW, eb, 0)

    @pl.when(wid < NEXTRA)
    def _():
        eb(RPW, 0)
    pltpu.sync_copy(accv, out_hbm.at[pl.ds(wid * N, N)])


def _tc_first_body(degp_ref, x_ref, w_ref, dis_ref, hp_ref):
    deg = jnp.sum(degp_ref[...], axis=0, keepdims=True) + 1.0
    dis_col = jnp.reshape(lax.rsqrt(deg), (N, 1))
    dis_ref[...] = dis_col
    h = jnp.dot(x_ref[...], w_ref[...], preferred_element_type=jnp.float32)
    hp_ref[...] = h * dis_col


_tc_first = pl.pallas_call(
    _tc_first_body,
    out_shape=(jax.ShapeDtypeStruct((N, 1), jnp.float32),
               jax.ShapeDtypeStruct((N, FH), jnp.float32)))


def _relu_comb(accp_ref, hp_ref, dis_ref, b_ref):
    acc = accp_ref[0, :N] + accp_ref[1, :N] + hp_ref[...]
    return jnp.maximum(acc * dis_ref[...] + b_ref[...], 0.0)


def _tc_mid_body(accp_ref, hp_ref, dis_ref, b_ref, w_ref, out_ref):
    o = _relu_comb(accp_ref, hp_ref, dis_ref, b_ref)
    out_ref[...] = jnp.dot(
        o, w_ref[...], preferred_element_type=jnp.float32) * dis_ref[...]


_tc_mid = pl.pallas_call(
    _tc_mid_body, out_shape=jax.ShapeDtypeStruct((N, FH), jnp.float32))


def _tc_mid3_body(accp_ref, hp_ref, dis_ref, b_ref, w_ref, out_ref):
    o = _relu_comb(accp_ref, hp_ref, dis_ref, b_ref)
    h3 = jnp.dot(o, w_ref[...], preferred_element_type=jnp.float32) \
        * dis_ref[...]
    out_ref[pl.ds(0, N)] = jnp.reshape(h3, (N,))
    out_ref[pl.ds(N, NP - N)] = jnp.zeros((NP - N,), jnp.float32)


_tc_mid3 = pl.pallas_call(
    _tc_mid3_body, out_shape=jax.ShapeDtypeStruct((NP,), jnp.float32))


def _tc_out_body(accp_ref, h3_ref, dis_ref, b_ref, out_ref):
    acc = jnp.sum(accp_ref[...], axis=0) + h3_ref[pl.ds(0, N)]
    out_ref[...] = jnp.reshape(acc, (N, 1)) * dis_ref[...] + b_ref[...]


_tc_out = pl.pallas_call(
    _tc_out_body, out_shape=jax.ShapeDtypeStruct((N, 1), jnp.float32))


def kernel(x, edge_index, edge_attr, W1, b1, W2, b2, W3, b3):
    ei = edge_index.astype(jnp.int32).reshape(2 * ER, CH)
    ew = edge_attr.astype(jnp.float32).reshape(ER, CH)
    degp = _sc_deg(ei, ew).reshape(NW, N)
    dis_col, h1p = _tc_first(degp, x, W1)
    acc1 = _sc_edge16(ei, ew, h1p)
    h2p = _tc_mid(acc1, h1p, dis_col, b1.reshape(1, FH), W2)
    acc2 = _sc_edge16(ei, ew, h2p)
    h3 = _tc_mid3(acc2, h2p, dis_col, b2.reshape(1, FH), W3)
    acc3 = _sc_edge1(ei, ew, h3).reshape(NW, N)
    return _tc_out(acc3, h3, dis_col, b3.reshape(1, 1))


# speedup vs baseline: 73.2254x; 1.0004x over previous
import functools

import jax
import jax.numpy as jnp
from jax import lax
from jax.experimental import pallas as pl
from jax.experimental.pallas import tpu as pltpu
from jax.experimental.pallas import tpu_sc as plsc

N = 10000
NP = 10240
FH = 16
NC = 2
NS = 16
NW = NC * NS
E = 320000
CH = 128
ER = E // CH
RPW = ER // NW
NEXTRA = ER % NW
N_PER_S = NP // NS
HR = N * FH // CH
HRS = HR // NS
NBUF = 6

_mesh = plsc.VectorSubcoreMesh(
    core_axis_name="c", subcore_axis_name="s", num_cores=NC, num_subcores=NS)

_sc_params = pltpu.CompilerParams(
    needs_layout_passes=False, use_tc_tiling_on_sc=False)


def _worker_id():
    return lax.axis_index("s") * NC + lax.axis_index("c")


def _stage_edges(ei_hbm, ew_hbm, wid, dstv, ewv, srcv=None):
    if srcv is not None:
        pltpu.sync_copy(ei_hbm.at[pl.ds(wid * RPW, RPW)],
                        srcv.at[pl.ds(0, RPW)])
    pltpu.sync_copy(ei_hbm.at[pl.ds(ER + wid * RPW, RPW)],
                    dstv.at[pl.ds(0, RPW)])
    pltpu.sync_copy(ew_hbm.at[pl.ds(wid * RPW, RPW)], ewv.at[pl.ds(0, RPW)])

    @pl.when(wid < NEXTRA)
    def _():
        if srcv is not None:
            pltpu.sync_copy(ei_hbm.at[pl.ds(NW * RPW + wid, 1)],
                            srcv.at[pl.ds(RPW, 1)])
        pltpu.sync_copy(ei_hbm.at[pl.ds(ER + NW * RPW + wid, 1)],
                        dstv.at[pl.ds(RPW, 1)])
        pltpu.sync_copy(ew_hbm.at[pl.ds(NW * RPW + wid, 1)],
                        ewv.at[pl.ds(RPW, 1)])


@functools.partial(
    pl.kernel,
    out_type=jax.ShapeDtypeStruct((NW * N,), jnp.float32),
    mesh=_mesh,
    compiler_params=_sc_params,
    scratch_types=[
        pltpu.VMEM((RPW + 1, CH), jnp.int32),
        pltpu.VMEM((RPW + 1, CH), jnp.float32),
        pltpu.VMEM((N,), jnp.float32),
    ],
)
def _sc_deg(ei_hbm, ew_hbm, out_hbm, dstv, ewv, accv):
    wid = _worker_id()
    _stage_edges(ei_hbm, ew_hbm, wid, dstv, ewv)
    z = jnp.zeros((16,), jnp.float32)

    def zb(i, carry):
        accv[pl.ds(i * 16, 16)] = z
        return carry
    lax.fori_loop(0, N // 16, zb, 0, unroll=8)

    def eb(g, carry):
        def ib(k, c2):
            idx = dstv[g, pl.ds(k * 16, 16)]
            w = ewv[g, pl.ds(k * 16, 16)]
            plsc.addupdate_scatter(accv, [idx], w)
            return c2
        return lax.fori_loop(0, CH // 16, ib, carry, unroll=8)
    lax.fori_loop(0, RPW, eb, 0)

    @pl.when(wid < NEXTRA)
    def _():
        eb(RPW, 0)
    pltpu.sync_copy(accv, out_hbm.at[pl.ds(wid * N, N)])


@functools.partial(
    pl.kernel,
    out_type=jax.ShapeDtypeStruct((NC, NP, FH), jnp.float32),
    mesh=_mesh,
    compiler_params=_sc_params,
    scratch_types=[
        pltpu.VMEM((RPW + 1, CH), jnp.int32),
        pltpu.VMEM((RPW + 1, CH), jnp.int32),
        pltpu.VMEM((RPW + 1, CH), jnp.float32),
        pltpu.VMEM((NBUF, CH, FH), jnp.float32),
        pltpu.VMEM((NBUF, CH, FH), jnp.float32),
        pltpu.VMEM_SHARED((NP, FH), jnp.float32),
        pltpu.VMEM_SHARED((N, FH), jnp.float32),
        pltpu.SemaphoreType.DMA((NBUF,)),
        pltpu.SemaphoreType.DMA((NBUF,)),
    ],
)
def _sc_edge16(ei_hbm, ew_hbm, hp_hbm, out_hbm,
               srcv, dstv, ewv, ga, sb, accs, hps, gsem, ssem):
    s = lax.axis_index("s")
    c = lax.axis_index("c")
    wid = s * NC + c
    _stage_edges(ei_hbm, ew_hbm, wid, dstv, ewv, srcv)

    nh = N // NS
    pltpu.sync_copy(hp_hbm.at[pl.ds(s * nh, nh)], hps.at[pl.ds(s * nh, nh)])

    z = jnp.zeros((16,), jnp.float32)

    def zb(i, carry):
        ga[0, i] = z
        return carry
    lax.fori_loop(0, CH, zb, 0, unroll=8)
    for q in range(N_PER_S // CH):
        pltpu.sync_copy(ga.at[0], accs.at[pl.ds(s * N_PER_S + q * CH, CH)])
    plsc.subcore_barrier()

    def start_gather(b, g):
        pltpu.async_copy(hps.at[srcv.at[g]], ga.at[b], gsem.at[b])

    def wait_gather(b, g):
        pltpu.make_async_copy(hps.at[srcv.at[g]], ga.at[b],
                              gsem.at[b]).wait()

    def start_scatter(b, g):
        pltpu.async_copy(sb.at[b], accs.at[dstv.at[g]], ssem.at[b],
                         add=True)

    def wait_scatter(b, g):
        pltpu.make_async_copy(sb.at[b], accs.at[dstv.at[g]],
                              ssem.at[b]).wait()

    def scale_group(b, g):
        def mb(j, c2):
            wv = ewv[g, pl.ds(j * 16, 16)]
            for t in range(16):
                sb[b, j * 16 + t] = ga[b, j * 16 + t] * wv[t]
            return c2
        lax.fori_loop(0, CH // 16, mb, 0, unroll=4)

    for b in range(NBUF):
        start_gather(b, b)

    def gb(o, carry):
        for b in range(NBUF):
            g = o * NBUF + b
            wait_gather(b, g)

            @pl.when(o > 0)
            def _():
                wait_scatter(b, g - NBUF)
            scale_group(b, g)
            start_scatter(b, g)

            @pl.when(o < RPW // NBUF - 1)
            def _():
                start_gather(b, g + NBUF)
        return carry
    lax.fori_loop(0, RPW // NBUF, gb, 0)
    for b in range(NBUF):
        wait_scatter(b, RPW - NBUF + b)

    @pl.when(wid < NEXTRA)
    def _():
        pltpu.async_copy(hps.at[srcv.at[RPW]], ga.at[0], gsem.at[0]).wait()
        scale_group(0, RPW)
        pltpu.async_copy(sb.at[0], accs.at[dstv.at[RPW]], ssem.at[0],
                         add=True).wait()

    plsc.subcore_barrier()
    pltpu.sync_copy(accs.at[pl.ds(s * N_PER_S, N_PER_S)],
                    out_hbm.at[c, pl.ds(s * N_PER_S, N_PER_S)])


@functools.partial(
    pl.kernel,
    out_type=jax.ShapeDtypeStruct((NW * N,), jnp.float32),
    mesh=_mesh,
    compiler_params=_sc_params,
    scratch_types=[
        pltpu.VMEM((RPW + 1, CH), jnp.int32),
        pltpu.VMEM((RPW + 1, CH), jnp.int32),
        pltpu.VMEM((RPW + 1, CH), jnp.float32),
        pltpu.VMEM((NP,), jnp.float32),
        pltpu.VMEM((N,), jnp.float32),
    ],
)
def _sc_edge1(ei_hbm, ew_hbm, h3_hbm, out_hbm,
              srcv, dstv, ewv, hv, accv):
    wid = _worker_id()
    _stage_edges(ei_hbm, ew_hbm, wid, dstv, ewv, srcv)
    pltpu.sync_copy(h3_hbm, hv)
    z = jnp.zeros((16,), jnp.float32)

    def zb(i, carry):
        accv[pl.ds(i * 16, 16)] = z
        return carry
    lax.fori_loop(0, N // 16, zb, 0, unroll=8)

    def eb(g, carry):
        def ib(k, c2):
            sl = pl.ds(k * 16, 16)
            vals = plsc.load_gather(hv, [srcv[g, sl]])
            plsc.addupdate_scatter(accv, [dstv[g, sl]], vals * ewv[g, sl])
            return c2
        return lax.fori_loop(0, CH // 16, ib, carry, unroll=8)
    lax.fori_loop(0, RPW, eb, 0)

    @pl.when(wid < NEXTRA)
    def _():
        eb(RPW, 0)
    pltpu.sync_copy(accv, out_hbm.at[pl.ds(wid * N, N)])


def _tc_h1_body(x_ref, w_ref, out_ref):
    out_ref[...] = jnp.dot(
        x_ref[...], w_ref[...], preferred_element_type=jnp.float32)


_tc_h1 = pl.pallas_call(
    _tc_h1_body, out_shape=jax.ShapeDtypeStruct((N, FH), jnp.float32))


def _tc_first_body(degp_ref, h_ref, dis_ref, hp_ref):
    deg = jnp.sum(degp_ref[...], axis=0, keepdims=True) + 1.0
    dis_col = jnp.reshape(lax.rsqrt(deg), (N, 1))
    dis_ref[...] = dis_col
    hp_ref[...] = h_ref[...] * dis_col


_tc_first = pl.pallas_call(
    _tc_first_body,
    out_shape=(jax.ShapeDtypeStruct((N, 1), jnp.float32),
               jax.ShapeDtypeStruct((N, FH), jnp.float32)))


def _relu_comb(accp_ref, hp_ref, dis_ref, b_ref):
    acc = accp_ref[0, :N] + accp_ref[1, :N] + hp_ref[...]
    return jnp.maximum(acc * dis_ref[...] + b_ref[...], 0.0)


def _tc_mid_body(accp_ref, hp_ref, dis_ref, b_ref, w_ref, out_ref):
    o = _relu_comb(accp_ref, hp_ref, dis_ref, b_ref)
    out_ref[...] = jnp.dot(
        o, w_ref[...], preferred_element_type=jnp.float32) * dis_ref[...]


_tc_mid = pl.pallas_call(
    _tc_mid_body, out_shape=jax.ShapeDtypeStruct((N, FH), jnp.float32))


def _tc_mid3_body(accp_ref, hp_ref, dis_ref, b_ref, w_ref, out_ref):
    o = _relu_comb(accp_ref, hp_ref, dis_ref, b_ref)
    h3 = jnp.dot(o, w_ref[...], preferred_element_type=jnp.float32) \
        * dis_ref[...]
    out_ref[pl.ds(0, N)] = jnp.reshape(h3, (N,))
    out_ref[pl.ds(N, NP - N)] = jnp.zeros((NP - N,), jnp.float32)


_tc_mid3 = pl.pallas_call(
    _tc_mid3_body, out_shape=jax.ShapeDtypeStruct((NP,), jnp.float32))


def _tc_out_body(accp_ref, h3_ref, dis_ref, b_ref, out_ref):
    acc = jnp.sum(accp_ref[...], axis=0) + h3_ref[pl.ds(0, N)]
    out_ref[...] = jnp.reshape(acc, (N, 1)) * dis_ref[...] + b_ref[...]


_tc_out = pl.pallas_call(
    _tc_out_body, out_shape=jax.ShapeDtypeStruct((N, 1), jnp.float32))


def kernel(x, edge_index, edge_attr, W1, b1, W2, b2, W3, b3):
    ei = edge_index.astype(jnp.int32).reshape(2 * ER, CH)
    ew = edge_attr.astype(jnp.float32).reshape(ER, CH)
    h1 = _tc_h1(x, W1)
    degp = _sc_deg(ei, ew).reshape(NW, N)
    dis_col, h1p = _tc_first(degp, h1)
    acc1 = _sc_edge16(ei, ew, h1p)
    h2p = _tc_mid(acc1, h1p, dis_col, b1.reshape(1, FH), W2)
    acc2 = _sc_edge16(ei, ew, h2p)
    h3 = _tc_mid3(acc2, h2p, dis_col, b2.reshape(1, FH), W3)
    acc3 = _sc_edge1(ei, ew, h3).reshape(NW, N)
    return _tc_out(acc3, h3, dis_col, b3.reshape(1, 1))
